# baseline (speedup 1.0000x reference)
import functools

import jax
import jax.numpy as jnp
from jax import lax
from jax.experimental import pallas as pl
from jax.experimental.pallas import tpu as pltpu
from jax.experimental.pallas import tpu_sc as plsc

LABEL_SIZE = 1000
HIDDEN = 768
EPS = 1e-12

NUM_CORES = 2
NUM_SUBCORES = 16
NW = NUM_CORES * NUM_SUBCORES

ROW_CHUNKS = 8
ROWS_PER_CHUNK = LABEL_SIZE // ROW_CHUNKS
BATCH_GROUPS = NW // ROW_CHUNKS
WAVE = 16


def _ln_body(w_ref, gamma_ref, beta_ref, out_ref):
    x = w_ref[...]
    mu = jnp.mean(x, axis=-1, keepdims=True)
    var = jnp.mean(jnp.square(x - mu), axis=-1, keepdims=True)
    out_ref[...] = (x - mu) * lax.rsqrt(var + EPS) * gamma_ref[...] + beta_ref[...]


def _layer_norm_table(W, gamma, beta):
    return pl.pallas_call(
        _ln_body,
        out_shape=jax.ShapeDtypeStruct((LABEL_SIZE, HIDDEN), jnp.float32),
    )(W, gamma, beta)


def _bcast_body(b_per_w, ln_hbm, out_hbm, buf, sem):
    wid = lax.axis_index("s") * NUM_CORES + lax.axis_index("c")
    rc = wid % ROW_CHUNKS
    bg = wid // ROW_CHUNKS
    row0 = rc * ROWS_PER_CHUNK
    b0 = bg * b_per_w

    pltpu.sync_copy(ln_hbm.at[pl.ds(row0, ROWS_PER_CHUNK), :], buf)

    for w0 in range(0, b_per_w, WAVE):
        nw = min(WAVE, b_per_w - w0)
        copies = [
            pltpu.async_copy(
                buf, out_hbm.at[b0 + w0 + j, pl.ds(row0, ROWS_PER_CHUNK), :], sem
            )
            for j in range(nw)
        ]
        for c in copies:
            c.wait()


def kernel(input_ids, W, gamma, beta):
    batch = input_ids.shape[0]
    assert batch % BATCH_GROUPS == 0
    b_per_w = batch // BATCH_GROUPS

    ln = _layer_norm_table(W, gamma, beta)

    mesh = plsc.VectorSubcoreMesh(core_axis_name="c", subcore_axis_name="s")
    bcast = functools.partial(
        pl.kernel,
        out_type=jax.ShapeDtypeStruct((batch, LABEL_SIZE, HIDDEN), jnp.float32),
        mesh=mesh,
        scratch_types=[
            pltpu.VMEM((ROWS_PER_CHUNK, HIDDEN), jnp.float32),
            pltpu.SemaphoreType.DMA,
        ],
        compiler_params=pltpu.CompilerParams(use_tc_tiling_on_sc=False),
    )(functools.partial(_bcast_body, b_per_w))
    return bcast(ln)


# speedup vs baseline: 3.9887x; 3.9887x over previous
import functools

import jax
import jax.numpy as jnp
from jax import lax
from jax.experimental import pallas as pl
from jax.experimental.pallas import tpu as pltpu
from jax.experimental.pallas import tpu_sc as plsc

LABEL_SIZE = 1000
HIDDEN = 768
EPS = 1e-12

NUM_CORES = 2
NUM_SUBCORES = 16
NW = NUM_CORES * NUM_SUBCORES

ROW_CHUNKS = 8
CHUNK = 128
LAST_CHUNK = LABEL_SIZE - (ROW_CHUNKS - 1) * CHUNK
BATCH_GROUPS = NW // ROW_CHUNKS
WAVE = 16


def _ln_body(w_ref, gamma_ref, beta_ref, out_ref):
    x = w_ref[...]
    mu = jnp.mean(x, axis=-1, keepdims=True)
    var = jnp.mean(jnp.square(x - mu), axis=-1, keepdims=True)
    out_ref[...] = (x - mu) * lax.rsqrt(var + EPS) * gamma_ref[...] + beta_ref[...]


def _layer_norm_table(W, gamma, beta):
    return pl.pallas_call(
        _ln_body,
        out_shape=jax.ShapeDtypeStruct((LABEL_SIZE, HIDDEN), jnp.float32),
    )(W, gamma, beta)


def _stream_out(buf_slice, out_hbm, row0, nrows, b0, b_per_w, sem):
    for w0 in range(0, b_per_w, WAVE):
        nw = min(WAVE, b_per_w - w0)
        copies = [
            pltpu.async_copy(
                buf_slice, out_hbm.at[b0 + w0 + j, pl.ds(row0, nrows), :], sem
            )
            for j in range(nw)
        ]
        for c in copies:
            c.wait()


def _bcast_body(b_per_w, ln_hbm, out_hbm, buf, sem):
    wid = lax.axis_index("s") * NUM_CORES + lax.axis_index("c")
    rc = wid % ROW_CHUNKS
    bg = wid // ROW_CHUNKS
    row0 = rc * CHUNK
    b0 = bg * b_per_w

    @pl.when(rc < ROW_CHUNKS - 1)
    def _():
        pltpu.sync_copy(ln_hbm.at[pl.ds(row0, CHUNK), :], buf)
        _stream_out(buf, out_hbm, row0, CHUNK, b0, b_per_w, sem)

    @pl.when(rc == ROW_CHUNKS - 1)
    def _():
        small = buf.at[pl.ds(0, LAST_CHUNK), :]
        pltpu.sync_copy(ln_hbm.at[pl.ds(row0, LAST_CHUNK), :], small)
        _stream_out(small, out_hbm, row0, LAST_CHUNK, b0, b_per_w, sem)


def kernel(input_ids, W, gamma, beta):
    batch = input_ids.shape[0]
    assert batch % BATCH_GROUPS == 0
    b_per_w = batch // BATCH_GROUPS

    ln = _layer_norm_table(W, gamma, beta)

    mesh = plsc.VectorSubcoreMesh(core_axis_name="c", subcore_axis_name="s")
    bcast = functools.partial(
        pl.kernel,
        out_type=jax.ShapeDtypeStruct((batch, LABEL_SIZE, HIDDEN), jnp.float32),
        mesh=mesh,
        scratch_types=[
            pltpu.VMEM((CHUNK, HIDDEN), jnp.float32),
            pltpu.SemaphoreType.DMA,
        ],
    )(functools.partial(_bcast_body, b_per_w))
    return bcast(ln)


# speedup vs baseline: 4.1465x; 1.0396x over previous
import functools

import jax
import jax.numpy as jnp
from jax import lax
from jax.experimental import pallas as pl
from jax.experimental.pallas import tpu as pltpu
from jax.experimental.pallas import tpu_sc as plsc

LABEL_SIZE = 1000
HIDDEN = 768
EPS = 1e-12

NUM_CORES = 2
NUM_SUBCORES = 16
NW = NUM_CORES * NUM_SUBCORES

ROW_CHUNKS = 8
CHUNK = 128
LAST_CHUNK = LABEL_SIZE - (ROW_CHUNKS - 1) * CHUNK
BATCH_GROUPS = NW // ROW_CHUNKS
WAVE = 64


def _ln_body(w_ref, gamma_ref, beta_ref, out_ref):
    x = w_ref[...]
    mu = jnp.mean(x, axis=-1, keepdims=True)
    var = jnp.mean(jnp.square(x - mu), axis=-1, keepdims=True)
    out_ref[...] = (x - mu) * lax.rsqrt(var + EPS) * gamma_ref[...] + beta_ref[...]


def _layer_norm_table(W, gamma, beta):
    return pl.pallas_call(
        _ln_body,
        out_shape=jax.ShapeDtypeStruct((LABEL_SIZE, HIDDEN), jnp.float32),
    )(W, gamma, beta)


def _stream_out(buf_slice, out_hbm, row0, nrows, b0, b_per_w, sem):
    for w0 in range(0, b_per_w, WAVE):
        nw = min(WAVE, b_per_w - w0)
        copies = [
            pltpu.async_copy(
                buf_slice, out_hbm.at[b0 + w0 + j, pl.ds(row0, nrows), :], sem
            )
            for j in range(nw)
        ]
        for c in copies:
            c.wait()


def _bcast_body(b_per_w, ln_hbm, out_hbm, buf, sem):
    wid = lax.axis_index("s") * NUM_CORES + lax.axis_index("c")
    rc = wid % ROW_CHUNKS
    bg = wid // ROW_CHUNKS
    row0 = rc * CHUNK
    b0 = bg * b_per_w

    @pl.when(rc < ROW_CHUNKS - 1)
    def _():
        pltpu.sync_copy(ln_hbm.at[pl.ds(row0, CHUNK), :], buf)
        _stream_out(buf, out_hbm, row0, CHUNK, b0, b_per_w, sem)

    @pl.when(rc == ROW_CHUNKS - 1)
    def _():
        small = buf.at[pl.ds(0, LAST_CHUNK), :]
        pltpu.sync_copy(ln_hbm.at[pl.ds(row0, LAST_CHUNK), :], small)
        _stream_out(small, out_hbm, row0, LAST_CHUNK, b0, b_per_w, sem)


def kernel(input_ids, W, gamma, beta):
    batch = input_ids.shape[0]
    assert batch % BATCH_GROUPS == 0
    b_per_w = batch // BATCH_GROUPS

    ln = _layer_norm_table(W, gamma, beta)

    mesh = plsc.VectorSubcoreMesh(core_axis_name="c", subcore_axis_name="s")
    bcast = functools.partial(
        pl.kernel,
        out_type=jax.ShapeDtypeStruct((batch, LABEL_SIZE, HIDDEN), jnp.float32),
        mesh=mesh,
        scratch_types=[
            pltpu.VMEM((CHUNK, HIDDEN), jnp.float32),
            pltpu.SemaphoreType.DMA,
        ],
    )(functools.partial(_bcast_body, b_per_w))
    return bcast(ln)
